# Initial kernel scaffold; baseline (speedup 1.0000x reference)
import jax
import jax.numpy as jnp
from jax.experimental import pallas as pl


def kernel(x, W, b):
    raise NotImplementedError("write your pallas kernel here")


# speedup vs baseline: 1.5760x; 1.5760x over previous
import jax
import jax.numpy as jnp
from jax import lax
from jax.experimental import pallas as pl
from jax.experimental.pallas import tpu as pltpu

INPUT_DIM = 768
NUM_EXPERTS = 64
BLOCK_TOKENS = 512


def _router_kernel(x_ref, w_ref, b_ref, gw_ref, idx_ref):
    x = x_ref[...]
    logits = lax.dot_general(
        x, w_ref[...],
        dimension_numbers=(((1,), (1,)), ((), ())),
        preferred_element_type=jnp.float32,
    ) + b_ref[...]
    m = jnp.max(logits, axis=-1, keepdims=True)
    e = jnp.exp(logits - m)
    w = e / jnp.sum(e, axis=-1, keepdims=True)
    gw_ref[...] = w
    idx_ref[0, 0, :] = jnp.argmax(w, axis=-1).astype(jnp.int32)


def kernel(x, W, b):
    B, S, D = x.shape
    T = B * S
    nblk = T // BLOCK_TOKENS
    xf = x.reshape(T, D)
    gw, idx = pl.pallas_call(
        _router_kernel,
        grid=(nblk,),
        in_specs=[
            pl.BlockSpec((BLOCK_TOKENS, D), lambda i: (i, 0)),
            pl.BlockSpec((NUM_EXPERTS, D), lambda i: (0, 0)),
            pl.BlockSpec((1, NUM_EXPERTS), lambda i: (0, 0)),
        ],
        out_specs=[
            pl.BlockSpec((BLOCK_TOKENS, NUM_EXPERTS), lambda i: (i, 0)),
            pl.BlockSpec((1, 1, BLOCK_TOKENS), lambda i: (i, 0, 0)),
        ],
        out_shape=[
            jax.ShapeDtypeStruct((T, NUM_EXPERTS), jnp.float32),
            jax.ShapeDtypeStruct((nblk, 1, BLOCK_TOKENS), jnp.int32),
        ],
        compiler_params=pltpu.CompilerParams(
            dimension_semantics=("parallel",),
        ),
    )(xf, W, b.reshape(1, NUM_EXPERTS))
    return gw.reshape(B, S, NUM_EXPERTS), idx.reshape(B, S)


# speedup vs baseline: 1.6953x; 1.0757x over previous
import jax
import jax.numpy as jnp
from jax import lax
from jax.experimental import pallas as pl
from jax.experimental.pallas import tpu as pltpu

INPUT_DIM = 768
NUM_EXPERTS = 64
BLOCK_TOKENS = 2048


def _router_kernel(x_ref, w_ref, b_ref, gw_ref, idx_ref):
    x = x_ref[...]
    logits = lax.dot_general(
        x, w_ref[...],
        dimension_numbers=(((1,), (1,)), ((), ())),
        preferred_element_type=jnp.float32,
    ) + b_ref[...]
    m = jnp.max(logits, axis=-1, keepdims=True)
    e = jnp.exp(logits - m)
    w = e / jnp.sum(e, axis=-1, keepdims=True)
    gw_ref[...] = w
    idx_ref[0, 0, :] = jnp.argmax(w, axis=-1).astype(jnp.int32)


def kernel(x, W, b):
    B, S, D = x.shape
    T = B * S
    nblk = T // BLOCK_TOKENS
    xf = x.reshape(T, D)
    gw, idx = pl.pallas_call(
        _router_kernel,
        grid=(nblk,),
        in_specs=[
            pl.BlockSpec((BLOCK_TOKENS, D), lambda i: (i, 0)),
            pl.BlockSpec((NUM_EXPERTS, D), lambda i: (0, 0)),
            pl.BlockSpec((1, NUM_EXPERTS), lambda i: (0, 0)),
        ],
        out_specs=[
            pl.BlockSpec((BLOCK_TOKENS, NUM_EXPERTS), lambda i: (i, 0)),
            pl.BlockSpec((1, 1, BLOCK_TOKENS), lambda i: (i, 0, 0)),
        ],
        out_shape=[
            jax.ShapeDtypeStruct((T, NUM_EXPERTS), jnp.float32),
            jax.ShapeDtypeStruct((nblk, 1, BLOCK_TOKENS), jnp.int32),
        ],
        compiler_params=pltpu.CompilerParams(
            dimension_semantics=("parallel",),
        ),
    )(xf, W, b.reshape(1, NUM_EXPERTS))
    return gw.reshape(B, S, NUM_EXPERTS), idx.reshape(B, S)


# speedup vs baseline: 2.0482x; 1.2082x over previous
import jax
import jax.numpy as jnp
from jax import lax
from jax.experimental import pallas as pl
from jax.experimental.pallas import tpu as pltpu

INPUT_DIM = 768
NUM_EXPERTS = 64
BLOCK_TOKENS = 4096


def _router_kernel(x_ref, w_ref, b_ref, gw_ref, idx_ref):
    x = x_ref[...]
    logits = lax.dot_general(
        x, w_ref[...],
        dimension_numbers=(((1,), (1,)), ((), ())),
        preferred_element_type=jnp.float32,
    ) + b_ref[...]
    m = jnp.max(logits, axis=-1, keepdims=True)
    e = jnp.exp(logits - m)
    w = e / jnp.sum(e, axis=-1, keepdims=True)
    gw_ref[...] = w
    idx_ref[0, 0, :] = jnp.argmax(w, axis=-1).astype(jnp.int32)


def kernel(x, W, b):
    B, S, D = x.shape
    T = B * S
    nblk = T // BLOCK_TOKENS
    xf = x.reshape(T, D)
    gw, idx = pl.pallas_call(
        _router_kernel,
        grid=(nblk,),
        in_specs=[
            pl.BlockSpec((BLOCK_TOKENS, D), lambda i: (i, 0)),
            pl.BlockSpec((NUM_EXPERTS, D), lambda i: (0, 0)),
            pl.BlockSpec((1, NUM_EXPERTS), lambda i: (0, 0)),
        ],
        out_specs=[
            pl.BlockSpec((BLOCK_TOKENS, NUM_EXPERTS), lambda i: (i, 0)),
            pl.BlockSpec((1, 1, BLOCK_TOKENS), lambda i: (i, 0, 0)),
        ],
        out_shape=[
            jax.ShapeDtypeStruct((T, NUM_EXPERTS), jnp.float32),
            jax.ShapeDtypeStruct((nblk, 1, BLOCK_TOKENS), jnp.int32),
        ],
        compiler_params=pltpu.CompilerParams(
            dimension_semantics=("parallel",),
        ),
    )(xf, W, b.reshape(1, NUM_EXPERTS))
    return gw.reshape(B, S, NUM_EXPERTS), idx.reshape(B, S)


# speedup vs baseline: 3.0378x; 1.4831x over previous
import jax
import jax.numpy as jnp
from jax import lax
from jax.experimental import pallas as pl
from jax.experimental.pallas import tpu as pltpu

INPUT_DIM = 768
NUM_EXPERTS = 64
BLOCK_COLS = 1024


def _router_kernel(x_ref, w_ref, b_ref, gw_ref, idx_ref):
    B, C, D = x_ref.shape
    x = x_ref[...].reshape(B * C, D)
    logits = lax.dot_general(
        x, w_ref[...],
        dimension_numbers=(((1,), (1,)), ((), ())),
        preferred_element_type=jnp.float32,
    ) + b_ref[...]
    m = jnp.max(logits, axis=-1, keepdims=True)
    e = jnp.exp(logits - m)
    w = e / jnp.sum(e, axis=-1, keepdims=True)
    gw_ref[...] = w.reshape(B, C, NUM_EXPERTS)
    ids = lax.broadcasted_iota(jnp.int32, logits.shape, 1)
    first_max = jnp.min(
        jnp.where(logits == m, ids, NUM_EXPERTS), axis=-1
    )
    idx_ref[...] = first_max.reshape(B, C)


def kernel(x, W, b):
    B, S, D = x.shape
    nblk = S // BLOCK_COLS
    gw, idx = pl.pallas_call(
        _router_kernel,
        grid=(nblk,),
        in_specs=[
            pl.BlockSpec((B, BLOCK_COLS, D), lambda i: (0, i, 0)),
            pl.BlockSpec((NUM_EXPERTS, D), lambda i: (0, 0)),
            pl.BlockSpec((1, NUM_EXPERTS), lambda i: (0, 0)),
        ],
        out_specs=[
            pl.BlockSpec((B, BLOCK_COLS, NUM_EXPERTS), lambda i: (0, i, 0)),
            pl.BlockSpec((B, BLOCK_COLS), lambda i: (0, i)),
        ],
        out_shape=[
            jax.ShapeDtypeStruct((B, S, NUM_EXPERTS), jnp.float32),
            jax.ShapeDtypeStruct((B, S), jnp.int32),
        ],
        compiler_params=pltpu.CompilerParams(
            dimension_semantics=("parallel",),
        ),
    )(x, W, b.reshape(1, NUM_EXPERTS))
    return gw, idx
